# Initial kernel scaffold; baseline (speedup 1.0000x reference)
import jax
import jax.numpy as jnp
from jax.experimental import pallas as pl


def kernel(inputs, table):
    raise NotImplementedError("write your pallas kernel here")


# speedup vs baseline: 1.4772x; 1.4772x over previous
import functools

import jax
import jax.numpy as jnp
from jax import lax
from jax.experimental import pallas as pl
from jax.experimental.pallas import tpu as pltpu
from jax.experimental.pallas import tpu_sc as plsc

_BATCH = 4096
_MAX_LEN = 200
_EMBED = 32
_B = _BATCH * _MAX_LEN
_NC = 2
_NS = 16
_NW = _NC * _NS
_BPW = _B // _NW
_CHUNK = 1600
_NCHUNK = _BPW // _CHUNK


@jax.jit
def _embedding_sc(idx_flat, table):
    mesh = plsc.VectorSubcoreMesh(core_axis_name="c", subcore_axis_name="s")

    @functools.partial(
        pl.kernel,
        mesh=mesh,
        out_type=jax.ShapeDtypeStruct((_B, _EMBED), jnp.float32),
        scratch_types=[
            pltpu.VMEM((_CHUNK,), jnp.int32),
            pltpu.VMEM((_CHUNK, _EMBED), jnp.float32),
            pltpu.SemaphoreType.DMA,
        ],
        compiler_params=pltpu.CompilerParams(use_tc_tiling_on_sc=False),
    )
    def k(idx_hbm, table_hbm, out_hbm, idx_v, rows_v, sem):
        wid = lax.axis_index("s") * _NC + lax.axis_index("c")
        base = wid * _BPW

        def body(i, carry):
            off = base + i * _CHUNK
            pltpu.sync_copy(idx_hbm.at[pl.ds(off, _CHUNK)], idx_v)
            pltpu.async_copy(table_hbm.at[idx_v], rows_v, sem).wait()
            pltpu.sync_copy(rows_v, out_hbm.at[pl.ds(off, _CHUNK)])
            return carry

        lax.fori_loop(0, _NCHUNK, body, 0)

    return k(idx_flat, table)


def kernel(inputs, table):
    idx_flat = inputs.reshape(-1).astype(jnp.int32)
    out = _embedding_sc(idx_flat, table)
    return out.reshape(_BATCH, _MAX_LEN, _EMBED)


# speedup vs baseline: 1.5035x; 1.0179x over previous
import functools

import jax
import jax.numpy as jnp
from jax import lax
from jax.experimental import pallas as pl
from jax.experimental.pallas import tpu as pltpu
from jax.experimental.pallas import tpu_sc as plsc

_BATCH = 4096
_MAX_LEN = 200
_EMBED = 32
_B = _BATCH * _MAX_LEN
_NC = 2
_NS = 16
_NW = _NC * _NS
_BPW = _B // _NW
_CHUNK = 1600
_NCHUNK = _BPW // _CHUNK


@jax.jit
def _embedding_sc(idx_flat, table):
    mesh = plsc.VectorSubcoreMesh(core_axis_name="c", subcore_axis_name="s")

    @functools.partial(
        pl.kernel,
        mesh=mesh,
        out_type=jax.ShapeDtypeStruct((_B, _EMBED), jnp.float32),
        scratch_types=[
            pltpu.VMEM((_BPW,), jnp.int32),
            pltpu.VMEM((2, _CHUNK, _EMBED), jnp.float32),
            pltpu.SemaphoreType.DMA((2,)),
            pltpu.SemaphoreType.DMA((2,)),
        ],
        compiler_params=pltpu.CompilerParams(use_tc_tiling_on_sc=False),
    )
    def k(idx_hbm, table_hbm, out_hbm, idx_v, rows_v, gsem, wsem):
        wid = lax.axis_index("s") * _NC + lax.axis_index("c")
        base = wid * _BPW
        pltpu.sync_copy(idx_hbm.at[pl.ds(base, _BPW)], idx_v)

        def g_desc(i, b):
            return pltpu.make_async_copy(
                table_hbm.at[idx_v.at[pl.ds(i * _CHUNK, _CHUNK)]],
                rows_v.at[b], gsem.at[b])

        def w_desc(i, b):
            return pltpu.make_async_copy(
                rows_v.at[b],
                out_hbm.at[pl.ds(base + i * _CHUNK, _CHUNK)], wsem.at[b])

        g_desc(0, 0).start()
        for i in range(_NCHUNK):
            b = i % 2
            nb = (i + 1) % 2
            if i + 1 < _NCHUNK:
                if i >= 1:
                    w_desc(i - 1, nb).wait()
                g_desc(i + 1, nb).start()
            g_desc(i, b).wait()
            w_desc(i, b).start()
        w_desc(_NCHUNK - 2, (_NCHUNK - 2) % 2).wait()
        w_desc(_NCHUNK - 1, (_NCHUNK - 1) % 2).wait()

    return k(idx_flat, table)


def kernel(inputs, table):
    idx_flat = inputs.reshape(-1).astype(jnp.int32)
    out = _embedding_sc(idx_flat, table)
    return out.reshape(_BATCH, _MAX_LEN, _EMBED)
